# Initial kernel scaffold; baseline (speedup 1.0000x reference)
import jax
import jax.numpy as jnp
from jax.experimental import pallas as pl


def kernel(prod_rule_idx_seq, edge_nodes, edge_symbols, node_symbols, atom_embed, bond_embed, W_layers, b_layers, W_out, b_out):
    raise NotImplementedError("write your pallas kernel here")


# speedup vs baseline: 2.6721x; 2.6721x over previous
import functools

import jax
import jax.numpy as jnp
from jax import lax
from jax.experimental import pallas as pl
from jax.experimental.pallas import tpu as pltpu
from jax.experimental.pallas import tpu_sc as plsc

_NUM_RULES = 1000
_NODES = 8
_EDGES = 12
_D = 64
_LAYERS = 3
_NSYM = 50
_TABLE_ROWS = 1024


def _table_body(esym, nsym, en, aemb, bemb, wt, bl, wot, bo, out, eh, nh, ea, na):
    R = _NUM_RULES
    f32 = jnp.float32
    iota = lax.broadcasted_iota(jnp.int32, (R, _NSYM), 1)
    for e in range(_EDGES):
        oh = (esym[:, e:e + 1] == iota).astype(f32)
        eh[pl.ds(e * R, R), :] = jnp.dot(oh, aemb[:, :], preferred_element_type=f32, precision=lax.Precision.HIGHEST)
    for n in range(_NODES):
        oh = (nsym[:, n:n + 1] == iota).astype(f32)
        nh[pl.ds(n * R, R), :] = jnp.dot(oh, bemb[:, :], preferred_element_type=f32, precision=lax.Precision.HIGHEST)

    def nslice(i):
        return pl.ds(pl.multiple_of(i * R, 8), R)

    ends = [(en[e, 0], en[e, 1]) for e in range(_EDGES)]
    v = None
    for l in range(_LAYERS):
        last = l == _LAYERS - 1
        if not last:
            for e in range(_EDGES):
                a, b = ends[e]
                ea[pl.ds(e * R, R), :] = (
                    eh[pl.ds(e * R, R), :] + nh[nslice(a), :] + nh[nslice(b), :]
                )
        na[:, :] = nh[:, :]
        for e in range(_EDGES):
            for k in range(2):
                i = ends[e][k]
                na[nslice(i), :] = na[nslice(i), :] + eh[pl.ds(e * R, R), :]
        if last:
            v = na[pl.ds((_NODES - 1) * R, R), :]
        else:
            w = wt[l]
            b = bl[l]
            eh[:, :] = jnp.maximum(jnp.dot(ea[:, :], w, preferred_element_type=f32, precision=lax.Precision.HIGHEST) + b, 0.0)
            nh[:, :] = jnp.maximum(jnp.dot(na[:, :], w, preferred_element_type=f32, precision=lax.Precision.HIGHEST) + b, 0.0)
    out[0:R, :] = jnp.tanh(jnp.dot(v, wot[:, :], preferred_element_type=f32, precision=lax.Precision.HIGHEST) + bo[:, :])
    out[pl.ds(R, _TABLE_ROWS - R), :] = jnp.zeros((_TABLE_ROWS - R, _D), f32)


def _build_table(esym, nsym, en, aemb, bemb, wt, bl, wot, bo):
    R = _NUM_RULES
    vmem = pl.BlockSpec(memory_space=pltpu.VMEM)
    return pl.pallas_call(
        _table_body,
        out_shape=jax.ShapeDtypeStruct((_TABLE_ROWS, _D), jnp.float32),
        in_specs=[
            vmem,
            vmem,
            pl.BlockSpec(memory_space=pltpu.SMEM),
            vmem,
            vmem,
            vmem,
            vmem,
            vmem,
            vmem,
        ],
        out_specs=pl.BlockSpec(memory_space=pltpu.VMEM),
        scratch_shapes=[
            pltpu.VMEM((_EDGES * R, _D), jnp.float32),
            pltpu.VMEM((_NODES * R, _D), jnp.float32),
            pltpu.VMEM((_EDGES * R, _D), jnp.float32),
            pltpu.VMEM((_NODES * R, _D), jnp.float32),
        ],
    )(esym, nsym, en, aemb, bemb, wt, bl, wot, bo)


@functools.lru_cache(maxsize=None)
def _gather_call(batch):
    info = plsc.get_sparse_core_info()
    nc, ns = info.num_cores, info.num_subcores
    nw = nc * ns
    bpw = batch // nw
    ch = 64
    nchunk = bpw // ch
    mesh = plsc.VectorSubcoreMesh(core_axis_name="c", subcore_axis_name="s")

    @functools.partial(
        pl.kernel,
        mesh=mesh,
        compiler_params=pltpu.CompilerParams(use_tc_tiling_on_sc=False),
        out_type=jax.ShapeDtypeStruct((batch, _D), jnp.float32),
        scratch_types=[
            pltpu.VMEM((nchunk, ch), jnp.int32),
            pltpu.VMEM((bpw, _D), jnp.float32),
            pltpu.SemaphoreType.DMA,
        ],
    )
    def gk(table_hbm, idx_hbm, out_hbm, idx_v, rows_v, sem):
        wid = lax.axis_index("s") * nc + lax.axis_index("c")
        pltpu.sync_copy(idx_hbm.at[wid], idx_v)
        copies = [
            pltpu.async_copy(
                table_hbm.at[idx_v.at[j]], rows_v.at[pl.ds(j * ch, ch)], sem
            )
            for j in range(nchunk)
        ]
        for c in copies:
            c.wait()
        pltpu.sync_copy(rows_v, out_hbm.at[pl.ds(wid * bpw, bpw)])

    return gk, nw, nchunk, ch


def kernel(prod_rule_idx_seq, edge_nodes, edge_symbols, node_symbols,
           atom_embed, bond_embed, W_layers, b_layers, W_out, b_out):
    table = _build_table(
        edge_symbols,
        node_symbols,
        edge_nodes.astype(jnp.int32),
        atom_embed,
        bond_embed,
        jnp.swapaxes(W_layers, 1, 2),
        b_layers.reshape(_LAYERS, 1, _D),
        W_out.T,
        b_out.reshape(1, _D),
    )
    bsz, length = prod_rule_idx_seq.shape
    batch = bsz * length
    gk, nw, nchunk, ch = _gather_call(batch)
    idx3 = prod_rule_idx_seq.reshape(nw, nchunk, ch).astype(jnp.int32)
    flat = gk(table, idx3)
    return flat.reshape(bsz, length, _D)


# speedup vs baseline: 3.1134x; 1.1651x over previous
import functools

import jax
import jax.numpy as jnp
from jax import lax
from jax.experimental import pallas as pl
from jax.experimental.pallas import tpu as pltpu
from jax.experimental.pallas import tpu_sc as plsc

_NUM_RULES = 1000
_NODES = 8
_EDGES = 12
_D = 64
_LAYERS = 3
_NSYM = 50
_PLANES = _EDGES + _NODES
_TABLE_ROWS = 1024


def _table_body(esym, nsym, en, emb, wt, bl, wot, bo, out, ohs, h, agg):
    R = _NUM_RULES
    f32 = jnp.float32
    iota = lax.broadcasted_iota(jnp.int32, (R, 128), 1)
    half = _PLANES // 2
    for hb in range(2):
        for q in range(half):
            p = hb * half + q
            if p < _EDGES:
                oh = (esym[:, p:p + 1] == iota).astype(f32)
            else:
                oh = (nsym[:, p - _EDGES:p - _EDGES + 1] + _NSYM == iota).astype(f32)
            ohs[pl.ds(q * R, R), :] = oh
        h[pl.ds(hb * half * R, half * R), :] = jnp.dot(
            ohs[:, :], emb[:, :], preferred_element_type=f32,
            precision=lax.Precision.HIGHEST)

    def nplane(i):
        return pl.ds(pl.multiple_of((_EDGES + i) * R, 8), R)

    node0 = _EDGES * R
    ends = [(en[e, 0], en[e, 1]) for e in range(_EDGES)]
    v = None
    for l in range(_LAYERS):
        last = l == _LAYERS - 1
        if not last:
            for e in range(_EDGES):
                a, b = ends[e]
                agg[pl.ds(e * R, R), :] = (
                    h[pl.ds(e * R, R), :] + h[nplane(a), :] + h[nplane(b), :]
                )
        agg[pl.ds(node0, _NODES * R), :] = h[pl.ds(node0, _NODES * R), :]
        for e in range(_EDGES):
            for k in range(2):
                i = ends[e][k]
                agg[nplane(i), :] = agg[nplane(i), :] + h[pl.ds(e * R, R), :]
        if last:
            v = agg[pl.ds(node0 + (_NODES - 1) * R, R), :]
        else:
            h[:, :] = jnp.maximum(
                jnp.dot(agg[:, :], wt[l], preferred_element_type=f32) + bl[l], 0.0)
    out[0:R, :] = jnp.tanh(jnp.dot(v, wot[:, :], preferred_element_type=f32) + bo[:, :])
    out[pl.ds(R, _TABLE_ROWS - R), :] = jnp.zeros((_TABLE_ROWS - R, _D), f32)


def _build_table(esym, nsym, en, emb, wt, bl, wot, bo):
    R = _NUM_RULES
    vmem = pl.BlockSpec(memory_space=pltpu.VMEM)
    return pl.pallas_call(
        _table_body,
        out_shape=jax.ShapeDtypeStruct((_TABLE_ROWS, _D), jnp.float32),
        in_specs=[
            vmem,
            vmem,
            pl.BlockSpec(memory_space=pltpu.SMEM),
            vmem,
            vmem,
            vmem,
            vmem,
            vmem,
        ],
        out_specs=pl.BlockSpec(memory_space=pltpu.VMEM),
        scratch_shapes=[
            pltpu.VMEM((_PLANES * R // 2, 128), jnp.float32),
            pltpu.VMEM((_PLANES * R, _D), jnp.float32),
            pltpu.VMEM((_PLANES * R, _D), jnp.float32),
        ],
    )(esym, nsym, en, emb, wt, bl, wot, bo)


@functools.lru_cache(maxsize=None)
def _gather_call(batch):
    info = plsc.get_sparse_core_info()
    nc, ns = info.num_cores, info.num_subcores
    nw = nc * ns
    bpw = batch // nw
    ch = 64
    nchunk = bpw // ch
    mesh = plsc.VectorSubcoreMesh(core_axis_name="c", subcore_axis_name="s")

    @functools.partial(
        pl.kernel,
        mesh=mesh,
        compiler_params=pltpu.CompilerParams(use_tc_tiling_on_sc=False),
        out_type=jax.ShapeDtypeStruct((batch, _D), jnp.float32),
        scratch_types=[
            pltpu.VMEM((nchunk, ch), jnp.int32),
            pltpu.VMEM((bpw, _D), jnp.float32),
            pltpu.SemaphoreType.DMA,
        ],
    )
    def gk(table_hbm, idx_hbm, out_hbm, idx_v, rows_v, sem):
        wid = lax.axis_index("s") * nc + lax.axis_index("c")
        pltpu.sync_copy(idx_hbm.at[wid], idx_v)
        copies = [
            pltpu.async_copy(
                table_hbm.at[idx_v.at[j]], rows_v.at[pl.ds(j * ch, ch)], sem
            )
            for j in range(nchunk)
        ]
        for c in copies:
            c.wait()
        pltpu.sync_copy(rows_v, out_hbm.at[pl.ds(wid * bpw, bpw)])

    return gk, nw, nchunk, ch


def kernel(prod_rule_idx_seq, edge_nodes, edge_symbols, node_symbols,
           atom_embed, bond_embed, W_layers, b_layers, W_out, b_out):
    emb = jnp.concatenate(
        [atom_embed, bond_embed,
         jnp.zeros((128 - 2 * _NSYM, _D), jnp.float32)], axis=0)
    table = _build_table(
        edge_symbols,
        node_symbols,
        edge_nodes.astype(jnp.int32),
        emb,
        jnp.swapaxes(W_layers, 1, 2),
        b_layers.reshape(_LAYERS, 1, _D),
        W_out.T,
        b_out.reshape(1, _D),
    )
    bsz, length = prod_rule_idx_seq.shape
    batch = bsz * length
    gk, nw, nchunk, ch = _gather_call(batch)
    idx3 = prod_rule_idx_seq.reshape(nw, nchunk, ch).astype(jnp.int32)
    flat = gk(table, idx3)
    return flat.reshape(bsz, length, _D)


# speedup vs baseline: 3.4485x; 1.1076x over previous
import functools

import jax
import jax.numpy as jnp
from jax import lax
from jax.experimental import pallas as pl
from jax.experimental.pallas import tpu as pltpu
from jax.experimental.pallas import tpu_sc as plsc

_NUM_RULES = 1000
_NODES = 8
_EDGES = 12
_D = 64
_LAYERS = 3
_NSYM = 50
_PLANES = _EDGES + _NODES
_TABLE_ROWS = 1024


def _split3(x):
    f32 = jnp.float32
    hi = x.astype(jnp.bfloat16).astype(f32)
    r = x - hi
    mid = r.astype(jnp.bfloat16).astype(f32)
    return hi, mid, r - mid


def _table_body(esym, nsym, en, aemb, bemb, wl, bl, wot, bo, out,
                ohs, h, agg, tab):
    R = _NUM_RULES
    f32 = jnp.float32
    dn = (((1,), (0,)), ((), ()))
    dnt = (((1,), (1,)), ((), ()))

    tab[:, :] = jnp.zeros((2 * _D, _D), f32)
    tab[0:_NSYM, :] = aemb[:, :]
    tab[pl.ds(_D, _NSYM), :] = bemb[:, :]
    iota = lax.broadcasted_iota(jnp.int32, (R, _D), 1)

    def onehot_embed(nplanes, table):
        p1, p2, p3 = _split3(table)
        o = ohs[pl.ds(0, nplanes * R), :]
        return (lax.dot_general(o, p1, dn, preferred_element_type=f32)
                + lax.dot_general(o, p2, dn, preferred_element_type=f32)
                + lax.dot_general(o, p3, dn, preferred_element_type=f32))

    for e in range(_EDGES):
        ohs[pl.ds(e * R, R), :] = (esym[:, e:e + 1] == iota).astype(f32)
    h[pl.ds(0, _EDGES * R), :] = onehot_embed(_EDGES, tab[0:_D, :])
    for n in range(_NODES):
        ohs[pl.ds(n * R, R), :] = (nsym[:, n:n + 1] == iota).astype(f32)
    h[pl.ds(_EDGES * R, _NODES * R), :] = onehot_embed(_NODES, tab[pl.ds(_D, _D), :])

    def nplane(i):
        return pl.ds(pl.multiple_of((_EDGES + i) * R, 8), R)

    node0 = _EDGES * R
    ends = [(en[e, 0], en[e, 1]) for e in range(_EDGES)]
    v = None
    for l in range(_LAYERS):
        last = l == _LAYERS - 1
        if not last:
            for e in range(_EDGES):
                a, b = ends[e]
                agg[pl.ds(e * R, R), :] = (
                    h[pl.ds(e * R, R), :] + h[nplane(a), :] + h[nplane(b), :]
                )
        agg[pl.ds(node0, _NODES * R), :] = h[pl.ds(node0, _NODES * R), :]
        for e in range(_EDGES):
            for k in range(2):
                i = ends[e][k]
                agg[nplane(i), :] = agg[nplane(i), :] + h[pl.ds(e * R, R), :]
        if last:
            v = agg[pl.ds(node0 + (_NODES - 1) * R, R), :]
        else:
            h[:, :] = jnp.maximum(
                lax.dot_general(agg[:, :], wl[l], dnt,
                                preferred_element_type=f32) + bl[l:l + 1, :],
                0.0)
    out[0:R, :] = jnp.tanh(
        lax.dot_general(v, wot[:, :], dnt, preferred_element_type=f32) + bo[:, :])
    out[pl.ds(R, _TABLE_ROWS - R), :] = jnp.zeros((_TABLE_ROWS - R, _D), f32)


def _build_table(esym, nsym, en, aemb, bemb, wl, bl, wot, bo):
    R = _NUM_RULES
    vmem = pl.BlockSpec(memory_space=pltpu.VMEM)
    return pl.pallas_call(
        _table_body,
        out_shape=jax.ShapeDtypeStruct((_TABLE_ROWS, _D), jnp.float32),
        in_specs=[
            vmem,
            vmem,
            pl.BlockSpec(memory_space=pltpu.SMEM),
            vmem,
            vmem,
            vmem,
            vmem,
            vmem,
            vmem,
        ],
        out_specs=pl.BlockSpec(memory_space=pltpu.VMEM),
        scratch_shapes=[
            pltpu.VMEM((_EDGES * R, _D), jnp.float32),
            pltpu.VMEM((_PLANES * R, _D), jnp.float32),
            pltpu.VMEM((_PLANES * R, _D), jnp.float32),
            pltpu.VMEM((2 * _D, _D), jnp.float32),
        ],
    )(esym, nsym, en, aemb, bemb, wl, bl, wot, bo)


@functools.lru_cache(maxsize=None)
def _gather_call(batch):
    info = plsc.get_sparse_core_info()
    nc, ns = info.num_cores, info.num_subcores
    nw = nc * ns
    bpw = batch // nw
    ch = 64
    nchunk = bpw // ch
    mesh = plsc.VectorSubcoreMesh(core_axis_name="c", subcore_axis_name="s")

    @functools.partial(
        pl.kernel,
        mesh=mesh,
        compiler_params=pltpu.CompilerParams(use_tc_tiling_on_sc=False),
        out_type=jax.ShapeDtypeStruct((batch, _D), jnp.float32),
        scratch_types=[
            pltpu.VMEM((nchunk, ch), jnp.int32),
            pltpu.VMEM((bpw, _D), jnp.float32),
            pltpu.SemaphoreType.DMA,
        ],
    )
    def gk(table_hbm, idx_hbm, out_hbm, idx_v, rows_v, sem):
        wid = lax.axis_index("s") * nc + lax.axis_index("c")
        pltpu.sync_copy(idx_hbm.at[wid], idx_v)
        copies = [
            pltpu.async_copy(
                table_hbm.at[idx_v.at[j]], rows_v.at[pl.ds(j * ch, ch)], sem
            )
            for j in range(nchunk)
        ]
        for c in copies:
            c.wait()
        pltpu.sync_copy(rows_v, out_hbm.at[pl.ds(wid * bpw, bpw)])

    return gk, nw, nchunk, ch


def kernel(prod_rule_idx_seq, edge_nodes, edge_symbols, node_symbols,
           atom_embed, bond_embed, W_layers, b_layers, W_out, b_out):
    table = _build_table(
        edge_symbols,
        node_symbols,
        edge_nodes.astype(jnp.int32),
        atom_embed,
        bond_embed,
        W_layers,
        b_layers,
        W_out,
        b_out.reshape(1, _D),
    )
    bsz, length = prod_rule_idx_seq.shape
    batch = bsz * length
    gk, nw, nchunk, ch = _gather_call(batch)
    idx3 = prod_rule_idx_seq.reshape(nw, nchunk, ch).astype(jnp.int32)
    flat = gk(table, idx3)
    return flat.reshape(bsz, length, _D)
